# Initial kernel scaffold; baseline (speedup 1.0000x reference)
import jax
import jax.numpy as jnp
from jax.experimental import pallas as pl


def kernel(x, W):
    raise NotImplementedError("write your pallas kernel here")


# speedup vs baseline: 2.3429x; 2.3429x over previous
import functools

import jax
import jax.numpy as jnp
from jax.experimental import pallas as pl

_TAU = 0.001
_BM = 512


def _body(x_ref, w_ref, o_ref):
    w = w_ref[...]
    m = jnp.max(w, axis=1, keepdims=True)
    e = jnp.exp((w - m) * (1.0 / _TAU))
    z = jnp.sum(e, axis=1)
    acc = jax.lax.dot_general(
        x_ref[...], e, (((1,), (1,)), ((), ())),
        preferred_element_type=jnp.float32)
    o_ref[...] = acc / z[None, :]


@jax.jit
def kernel(x, W):
    batch, in_dim = x.shape
    out_dim = W.shape[0]
    grid = (out_dim // _BM,)
    return pl.pallas_call(
        _body,
        grid=grid,
        in_specs=[
            pl.BlockSpec((batch, in_dim), lambda i: (0, 0)),
            pl.BlockSpec((_BM, in_dim), lambda i: (i, 0)),
        ],
        out_specs=pl.BlockSpec((batch, _BM), lambda i: (0, i)),
        out_shape=jax.ShapeDtypeStruct((batch, out_dim), jnp.float32),
    )(x, W)
